# baseline (speedup 1.0000x reference)
import functools

import jax
import jax.numpy as jnp
from jax import lax
from jax.experimental import pallas as pl
from jax.experimental.pallas import tpu as pltpu
from jax.experimental.pallas import tpu_sc as plsc

_N = 100000
_E = 3200000
_H = 16
_VIN = 128
_UIN = 64
_G = 16

_EB = 128
_NBLK = _E // _EB
_CB = 8
_CE = _CB * _EB
_NCHUNK = _NBLK // _CB
_NC = 2
_NS = 16
_NW = _NC * _NS
_ITERS = -(-_NCHUNK // _NW)
_RA = 6256
_RLAST = _N - 15 * _RA

_BN = 4000
_NB = _N // _BN

_XB = 6400
_XG = _E // _XB


def _xpose_body(in_ref, out_ref):
    t = in_ref[...].T
    t3 = t.reshape(_XB // 8, 8, _H)
    for v in range(8):
        out_ref[:, 16 * v:16 * (v + 1)] = t3[:, v, :]


def _tc_xpose(eaT):
    return pl.pallas_call(
        _xpose_body,
        grid=(_XG,),
        in_specs=[pl.BlockSpec((_H, _XB), lambda i: (0, i))],
        out_specs=pl.BlockSpec((_XB // 8, 128), lambda i: (i, 0)),
        out_shape=jax.ShapeDtypeStruct((_E // 8, 128), jnp.float32),
        compiler_params=pltpu.CompilerParams(
            dimension_semantics=("arbitrary",)),
    )(eaT)


def _sc_body(col_hbm, ea_hbm, z_hbm, out_hbm, idx_v, val_v, agg_sh):
    c = lax.axis_index("c")
    s = lax.axis_index("s")
    wid = s * _NC + c
    rowbase = s * _RA

    @pl.when(s < _NS - 1)
    def _():
        pltpu.sync_copy(z_hbm.at[pl.ds(rowbase, _RA)],
                        agg_sh.at[pl.ds(rowbase, _RA)])

    @pl.when(s == _NS - 1)
    def _():
        pltpu.sync_copy(z_hbm.at[pl.ds(rowbase, _RLAST)],
                        agg_sh.at[pl.ds(rowbase, _RLAST)])

    plsc.subcore_barrier()

    def body(it, carry):
        chunk = it * _NW + wid

        @pl.when(chunk < _NCHUNK)
        def _():
            pltpu.sync_copy(col_hbm.at[pl.ds(chunk * _CB, _CB)], idx_v)
            pltpu.sync_copy(ea_hbm.at[pl.ds(chunk * _CE, _CE)], val_v)
            for j in range(_CB):
                pltpu.sync_copy(val_v.at[pl.ds(j * _EB, _EB)],
                                agg_sh.at[idx_v.at[j]], add=True)

        return carry

    lax.fori_loop(0, _ITERS, body, 0)
    plsc.subcore_barrier()

    @pl.when(s < _NS - 1)
    def _():
        pltpu.sync_copy(agg_sh.at[pl.ds(rowbase, _RA)],
                        out_hbm.at[pl.ds(c * _N + rowbase, _RA)])

    @pl.when(s == _NS - 1)
    def _():
        pltpu.sync_copy(agg_sh.at[pl.ds(rowbase, _RLAST)],
                        out_hbm.at[pl.ds(c * _N + rowbase, _RLAST)])


@functools.cache
def _sc_scatter_add():
    return pl.kernel(
        _sc_body,
        out_type=jax.ShapeDtypeStruct((_NC * _N, _H), jnp.float32),
        mesh=plsc.VectorSubcoreMesh(core_axis_name="c", subcore_axis_name="s",
                                    num_cores=_NC, num_subcores=_NS),
        scratch_types=[
            pltpu.VMEM((_CB, _EB), jnp.int32),
            pltpu.VMEM((_CE, _H), jnp.float32),
            pltpu.VMEM_SHARED((_N, _H), jnp.float32),
        ],
        compiler_params=pltpu.CompilerParams(use_tc_tiling_on_sc=False),
    )


def _mlp_x_body(x_ref, b_ref, u_ref, W1_ref, b1_ref, o_ref):
    f32 = jnp.float32
    dotT = lambda a, b: lax.dot_general(a, b, (((1,), (1,)), ((), ())),
                                        preferred_element_type=f32)
    W1 = W1_ref[...]
    uh = dotT(u_ref[...], W1[:, _VIN + _H:]) + b1_ref[...]
    bidx = b_ref[0, 0, :]
    oh = (bidx[:, None] == lax.broadcasted_iota(jnp.int32, (1, _G), 1)
          ).astype(f32)
    acc = dotT(x_ref[...], W1[:, :_VIN])
    o_ref[...] = acc + lax.dot_general(oh, uh, (((1,), (0,)), ((), ())),
                                       preferred_element_type=f32)


def _tc_mlp_x(x, batch3, u, W1, b1r):
    return pl.pallas_call(
        _mlp_x_body,
        grid=(_NB,),
        in_specs=[
            pl.BlockSpec((_BN, _VIN), lambda i: (i, 0)),
            pl.BlockSpec((1, 1, _BN), lambda i: (i, 0, 0)),
            pl.BlockSpec((_G, _UIN), lambda i: (0, 0)),
            pl.BlockSpec((_H, _VIN + _H + _UIN), lambda i: (0, 0)),
            pl.BlockSpec((1, _H), lambda i: (0, 0)),
        ],
        out_specs=pl.BlockSpec((_BN, _H), lambda i: (i, 0)),
        out_shape=jax.ShapeDtypeStruct((_N, _H), jnp.float32),
        compiler_params=pltpu.CompilerParams(
            dimension_semantics=("arbitrary",)),
    )(x, batch3, u, W1, b1r)


def _mlp_fin_body(a_ref, p0_ref, p1_ref, W1_ref, W2_ref, b2_ref, o_ref):
    f32 = jnp.float32
    dotT = lambda a, b: lax.dot_general(a, b, (((1,), (1,)), ((), ())),
                                        preferred_element_type=f32)
    W1a = W1_ref[...][:, _VIN:_VIN + _H]
    acc = a_ref[...] + dotT(p0_ref[...] + p1_ref[...], W1a)
    h = jnp.maximum(acc, 0.0)
    o_ref[...] = dotT(h, W2_ref[...]) + b2_ref[...]


def _tc_mlp_fin(accx, partial, W1, W2, b2r):
    return pl.pallas_call(
        _mlp_fin_body,
        grid=(_NB,),
        in_specs=[
            pl.BlockSpec((_BN, _H), lambda i: (i, 0)),
            pl.BlockSpec((_BN, _H), lambda i: (i, 0)),
            pl.BlockSpec((_BN, _H), lambda i: (i + _NB, 0)),
            pl.BlockSpec((_H, _VIN + _H + _UIN), lambda i: (0, 0)),
            pl.BlockSpec((_H, _H), lambda i: (0, 0)),
            pl.BlockSpec((1, _H), lambda i: (0, 0)),
        ],
        out_specs=pl.BlockSpec((_BN, _H), lambda i: (i, 0)),
        out_shape=jax.ShapeDtypeStruct((_N, _H), jnp.float32),
        compiler_params=pltpu.CompilerParams(
            dimension_semantics=("arbitrary",)),
    )(accx, partial, partial, W1, W2, b2r)


def kernel(x, edge_index, edge_attr, u, batch, W1, b1, W2, b2):
    col2d = edge_index[1].astype(jnp.int32).reshape(_NBLK, _EB)
    zeros = jnp.zeros((_N, _H), jnp.float32)
    ea128 = _tc_xpose(edge_attr.T)
    partial = _sc_scatter_add()(col2d, ea128.reshape(_E, _H), zeros)
    batch3 = batch.astype(jnp.int32).reshape(_NB, 1, _BN)
    accx = _tc_mlp_x(x, batch3, u, W1, b1.reshape(1, _H))
    return _tc_mlp_fin(accx, partial, W1, W2, b2.reshape(1, _H))


# speedup vs baseline: 1.0606x; 1.0606x over previous
import functools

import jax
import jax.numpy as jnp
from jax import lax
from jax.experimental import pallas as pl
from jax.experimental.pallas import tpu as pltpu
from jax.experimental.pallas import tpu_sc as plsc

_N = 100000
_E = 3200000
_H = 16
_VIN = 128
_UIN = 64
_G = 16

_EB = 128
_NBLK = _E // _EB
_CB = 8
_CE = _CB * _EB
_NCHUNK = _NBLK // _CB
_NC = 2
_NS = 16
_NW = _NC * _NS
_ITERS = -(-_NCHUNK // _NW)
_RA = 6256
_RLAST = _N - 15 * _RA

_BN = 4000
_NB = _N // _BN

_XB = 5120
_XG = _E // _XB
_XG1 = 313
_XG2 = _XG - _XG1
_E1 = _XG1 * _XB
_E2 = _E - _E1
_NCH1 = _E1 // _CE
_NCH2 = _E2 // _CE


def _xpose_body(in_ref, out_ref):
    t = in_ref[...].T
    t3 = t.reshape(_XB // 8, 8, _H)
    for v in range(8):
        out_ref[:, 16 * v:16 * (v + 1)] = t3[:, v, :]


def _tc_xpose(eaT, nblk, blk0):
    return pl.pallas_call(
        _xpose_body,
        grid=(nblk,),
        in_specs=[pl.BlockSpec((_H, _XB), lambda i: (0, i + blk0))],
        out_specs=pl.BlockSpec((_XB // 8, 128), lambda i: (i, 0)),
        out_shape=jax.ShapeDtypeStruct((nblk * _XB // 8, 128), jnp.float32),
        compiler_params=pltpu.CompilerParams(
            dimension_semantics=("arbitrary",)),
    )(eaT)


def _make_sc_body(nchunk):
    iters = -(-nchunk // _NW)

    def _sc_body(col_hbm, ea_hbm, z_hbm, out_hbm, idx_v, val_v, agg_sh):
        c = lax.axis_index("c")
        s = lax.axis_index("s")
        wid = s * _NC + c
        rowbase = s * _RA

        @pl.when(s < _NS - 1)
        def _():
            pltpu.sync_copy(z_hbm.at[pl.ds(rowbase, _RA)],
                            agg_sh.at[pl.ds(rowbase, _RA)])

        @pl.when(s == _NS - 1)
        def _():
            pltpu.sync_copy(z_hbm.at[pl.ds(rowbase, _RLAST)],
                            agg_sh.at[pl.ds(rowbase, _RLAST)])

        plsc.subcore_barrier()

        def body(it, carry):
            chunk = it * _NW + wid

            @pl.when(chunk < nchunk)
            def _():
                pltpu.sync_copy(col_hbm.at[pl.ds(chunk * _CB, _CB)], idx_v)
                pltpu.sync_copy(ea_hbm.at[pl.ds(chunk * _CE, _CE)], val_v)
                for j in range(_CB):
                    pltpu.sync_copy(val_v.at[pl.ds(j * _EB, _EB)],
                                    agg_sh.at[idx_v.at[j]], add=True)

            return carry

        lax.fori_loop(0, iters, body, 0)
        plsc.subcore_barrier()

        @pl.when(s < _NS - 1)
        def _():
            pltpu.sync_copy(agg_sh.at[pl.ds(rowbase, _RA)],
                            out_hbm.at[pl.ds(c * _N + rowbase, _RA)])

        @pl.when(s == _NS - 1)
        def _():
            pltpu.sync_copy(agg_sh.at[pl.ds(rowbase, _RLAST)],
                            out_hbm.at[pl.ds(c * _N + rowbase, _RLAST)])

    return _sc_body


@functools.cache
def _sc_scatter_add(nchunk):
    return pl.kernel(
        _make_sc_body(nchunk),
        out_type=jax.ShapeDtypeStruct((_NC * _N, _H), jnp.float32),
        mesh=plsc.VectorSubcoreMesh(core_axis_name="c", subcore_axis_name="s",
                                    num_cores=_NC, num_subcores=_NS),
        scratch_types=[
            pltpu.VMEM((_CB, _EB), jnp.int32),
            pltpu.VMEM((_CE, _H), jnp.float32),
            pltpu.VMEM_SHARED((_N, _H), jnp.float32),
        ],
        compiler_params=pltpu.CompilerParams(use_tc_tiling_on_sc=False),
    )


def _mlp_x_body(x_ref, b_ref, u_ref, W1_ref, b1_ref, o_ref):
    f32 = jnp.float32
    dotT = lambda a, b: lax.dot_general(a, b, (((1,), (1,)), ((), ())),
                                        preferred_element_type=f32)
    W1 = W1_ref[...]
    uh = dotT(u_ref[...], W1[:, _VIN + _H:]) + b1_ref[...]
    bidx = b_ref[0, 0, :]
    oh = (bidx[:, None] == lax.broadcasted_iota(jnp.int32, (1, _G), 1)
          ).astype(f32)
    acc = dotT(x_ref[...], W1[:, :_VIN])
    o_ref[...] = acc + lax.dot_general(oh, uh, (((1,), (0,)), ((), ())),
                                       preferred_element_type=f32)


def _tc_mlp_x(x, batch3, u, W1, b1r):
    return pl.pallas_call(
        _mlp_x_body,
        grid=(_NB,),
        in_specs=[
            pl.BlockSpec((_BN, _VIN), lambda i: (i, 0)),
            pl.BlockSpec((1, 1, _BN), lambda i: (i, 0, 0)),
            pl.BlockSpec((_G, _UIN), lambda i: (0, 0)),
            pl.BlockSpec((_H, _VIN + _H + _UIN), lambda i: (0, 0)),
            pl.BlockSpec((1, _H), lambda i: (0, 0)),
        ],
        out_specs=pl.BlockSpec((_BN, _H), lambda i: (i, 0)),
        out_shape=jax.ShapeDtypeStruct((_N, _H), jnp.float32),
        compiler_params=pltpu.CompilerParams(
            dimension_semantics=("arbitrary",)),
    )(x, batch3, u, W1, b1r)


def _mlp_fin_body(a_ref, p0_ref, p1_ref, p2_ref, p3_ref, W1_ref, W2_ref,
                  b2_ref, o_ref):
    f32 = jnp.float32
    dotT = lambda a, b: lax.dot_general(a, b, (((1,), (1,)), ((), ())),
                                        preferred_element_type=f32)
    W1a = W1_ref[...][:, _VIN:_VIN + _H]
    p = (p0_ref[...] + p1_ref[...]) + (p2_ref[...] + p3_ref[...])
    acc = a_ref[...] + dotT(p, W1a)
    h = jnp.maximum(acc, 0.0)
    o_ref[...] = dotT(h, W2_ref[...]) + b2_ref[...]


def _tc_mlp_fin(accx, pa, pb, W1, W2, b2r):
    return pl.pallas_call(
        _mlp_fin_body,
        grid=(_NB,),
        in_specs=[
            pl.BlockSpec((_BN, _H), lambda i: (i, 0)),
            pl.BlockSpec((_BN, _H), lambda i: (i, 0)),
            pl.BlockSpec((_BN, _H), lambda i: (i + _NB, 0)),
            pl.BlockSpec((_BN, _H), lambda i: (i, 0)),
            pl.BlockSpec((_BN, _H), lambda i: (i + _NB, 0)),
            pl.BlockSpec((_H, _VIN + _H + _UIN), lambda i: (0, 0)),
            pl.BlockSpec((_H, _H), lambda i: (0, 0)),
            pl.BlockSpec((1, _H), lambda i: (0, 0)),
        ],
        out_specs=pl.BlockSpec((_BN, _H), lambda i: (i, 0)),
        out_shape=jax.ShapeDtypeStruct((_N, _H), jnp.float32),
        compiler_params=pltpu.CompilerParams(
            dimension_semantics=("arbitrary",)),
    )(accx, pa, pa, pb, pb, W1, W2, b2r)


def kernel(x, edge_index, edge_attr, u, batch, W1, b1, W2, b2):
    col2d = edge_index[1].astype(jnp.int32).reshape(_NBLK, _EB)
    zeros = jnp.zeros((_N, _H), jnp.float32)
    eaT = edge_attr.T
    ea1 = _tc_xpose(eaT, _XG1, 0)
    pa = _sc_scatter_add(_NCH1)(col2d[:_NCH1 * _CB],
                                ea1.reshape(_E1, _H), zeros)
    ea2 = _tc_xpose(eaT, _XG2, _XG1)
    pb = _sc_scatter_add(_NCH2)(col2d[_NCH1 * _CB:],
                                ea2.reshape(_E2, _H), zeros)
    batch3 = batch.astype(jnp.int32).reshape(_NB, 1, _BN)
    accx = _tc_mlp_x(x, batch3, u, W1, b1.reshape(1, _H))
    return _tc_mlp_fin(accx, pa, pb, W1, W2, b2.reshape(1, _H))


# speedup vs baseline: 1.1074x; 1.0441x over previous
import functools

import jax
import jax.numpy as jnp
from jax import lax
from jax.experimental import pallas as pl
from jax.experimental.pallas import tpu as pltpu
from jax.experimental.pallas import tpu_sc as plsc

_N = 100000
_E = 3200000
_H = 16
_VIN = 128
_UIN = 64
_G = 16

_EB = 128
_NBLK = _E // _EB
_CB = 8
_CE = _CB * _EB
_NCHUNK = _NBLK // _CB
_NC = 2
_NS = 16
_NW = _NC * _NS
_ITERS = -(-_NCHUNK // _NW)
_RA = 6256
_RLAST = _N - 15 * _RA

_BN = 4000
_NB = _N // _BN

_XB = 5120
_XG = _E // _XB
_XG1 = 438
_XG2 = _XG - _XG1
_E1 = _XG1 * _XB
_E2 = _E - _E1
_NCH1 = _E1 // _CE
_NCH2 = _E2 // _CE


def _xpose_body(in_ref, out_ref):
    t = in_ref[...].T
    t3 = t.reshape(_XB // 8, 8, _H)
    for v in range(8):
        out_ref[:, 16 * v:16 * (v + 1)] = t3[:, v, :]


def _tc_xpose(eaT, nblk, blk0):
    return pl.pallas_call(
        _xpose_body,
        grid=(nblk,),
        in_specs=[pl.BlockSpec((_H, _XB), lambda i: (0, i + blk0))],
        out_specs=pl.BlockSpec((_XB // 8, 128), lambda i: (i, 0)),
        out_shape=jax.ShapeDtypeStruct((nblk * _XB // 8, 128), jnp.float32),
        compiler_params=pltpu.CompilerParams(
            dimension_semantics=("arbitrary",)),
    )(eaT)


def _make_sc_body(nchunk):
    iters = -(-nchunk // _NW)

    def _sc_body(col_hbm, ea_hbm, z_hbm, out_hbm, idx_v, val_v, agg_sh):
        c = lax.axis_index("c")
        s = lax.axis_index("s")
        wid = s * _NC + c
        rowbase = s * _RA

        @pl.when(s < _NS - 1)
        def _():
            pltpu.sync_copy(z_hbm.at[pl.ds(rowbase, _RA)],
                            agg_sh.at[pl.ds(rowbase, _RA)])

        @pl.when(s == _NS - 1)
        def _():
            pltpu.sync_copy(z_hbm.at[pl.ds(rowbase, _RLAST)],
                            agg_sh.at[pl.ds(rowbase, _RLAST)])

        plsc.subcore_barrier()

        def body(it, carry):
            chunk = it * _NW + wid

            @pl.when(chunk < nchunk)
            def _():
                pltpu.sync_copy(col_hbm.at[pl.ds(chunk * _CB, _CB)], idx_v)
                pltpu.sync_copy(ea_hbm.at[pl.ds(chunk * _CE, _CE)], val_v)
                for j in range(_CB):
                    pltpu.sync_copy(val_v.at[pl.ds(j * _EB, _EB)],
                                    agg_sh.at[idx_v.at[j]], add=True)

            return carry

        lax.fori_loop(0, iters, body, 0)
        plsc.subcore_barrier()

        @pl.when(s < _NS - 1)
        def _():
            pltpu.sync_copy(agg_sh.at[pl.ds(rowbase, _RA)],
                            out_hbm.at[pl.ds(c * _N + rowbase, _RA)])

        @pl.when(s == _NS - 1)
        def _():
            pltpu.sync_copy(agg_sh.at[pl.ds(rowbase, _RLAST)],
                            out_hbm.at[pl.ds(c * _N + rowbase, _RLAST)])

    return _sc_body


@functools.cache
def _sc_scatter_add(nchunk):
    return pl.kernel(
        _make_sc_body(nchunk),
        out_type=jax.ShapeDtypeStruct((_NC * _N, _H), jnp.float32),
        mesh=plsc.VectorSubcoreMesh(core_axis_name="c", subcore_axis_name="s",
                                    num_cores=_NC, num_subcores=_NS),
        scratch_types=[
            pltpu.VMEM((_CB, _EB), jnp.int32),
            pltpu.VMEM((_CE, _H), jnp.float32),
            pltpu.VMEM_SHARED((_N, _H), jnp.float32),
        ],
        compiler_params=pltpu.CompilerParams(use_tc_tiling_on_sc=False),
    )


def _mlp_x_body(x_ref, b_ref, u_ref, W1_ref, b1_ref, o_ref):
    f32 = jnp.float32
    dotT = lambda a, b: lax.dot_general(a, b, (((1,), (1,)), ((), ())),
                                        preferred_element_type=f32)
    W1 = W1_ref[...]
    uh = dotT(u_ref[...], W1[:, _VIN + _H:]) + b1_ref[...]
    bidx = b_ref[0, 0, :]
    oh = (bidx[:, None] == lax.broadcasted_iota(jnp.int32, (1, _G), 1)
          ).astype(f32)
    acc = dotT(x_ref[...], W1[:, :_VIN])
    o_ref[...] = acc + lax.dot_general(oh, uh, (((1,), (0,)), ((), ())),
                                       preferred_element_type=f32)


def _tc_mlp_x(x, batch3, u, W1, b1r):
    return pl.pallas_call(
        _mlp_x_body,
        grid=(_NB,),
        in_specs=[
            pl.BlockSpec((_BN, _VIN), lambda i: (i, 0)),
            pl.BlockSpec((1, 1, _BN), lambda i: (i, 0, 0)),
            pl.BlockSpec((_G, _UIN), lambda i: (0, 0)),
            pl.BlockSpec((_H, _VIN + _H + _UIN), lambda i: (0, 0)),
            pl.BlockSpec((1, _H), lambda i: (0, 0)),
        ],
        out_specs=pl.BlockSpec((_BN, _H), lambda i: (i, 0)),
        out_shape=jax.ShapeDtypeStruct((_N, _H), jnp.float32),
        compiler_params=pltpu.CompilerParams(
            dimension_semantics=("arbitrary",)),
    )(x, batch3, u, W1, b1r)


def _mlp_fin_body(a_ref, p0_ref, p1_ref, p2_ref, p3_ref, W1_ref, W2_ref,
                  b2_ref, o_ref):
    f32 = jnp.float32
    dotT = lambda a, b: lax.dot_general(a, b, (((1,), (1,)), ((), ())),
                                        preferred_element_type=f32)
    W1a = W1_ref[...][:, _VIN:_VIN + _H]
    p = (p0_ref[...] + p1_ref[...]) + (p2_ref[...] + p3_ref[...])
    acc = a_ref[...] + dotT(p, W1a)
    h = jnp.maximum(acc, 0.0)
    o_ref[...] = dotT(h, W2_ref[...]) + b2_ref[...]


def _tc_mlp_fin(accx, pa, pb, W1, W2, b2r):
    return pl.pallas_call(
        _mlp_fin_body,
        grid=(_NB,),
        in_specs=[
            pl.BlockSpec((_BN, _H), lambda i: (i, 0)),
            pl.BlockSpec((_BN, _H), lambda i: (i, 0)),
            pl.BlockSpec((_BN, _H), lambda i: (i + _NB, 0)),
            pl.BlockSpec((_BN, _H), lambda i: (i, 0)),
            pl.BlockSpec((_BN, _H), lambda i: (i + _NB, 0)),
            pl.BlockSpec((_H, _VIN + _H + _UIN), lambda i: (0, 0)),
            pl.BlockSpec((_H, _H), lambda i: (0, 0)),
            pl.BlockSpec((1, _H), lambda i: (0, 0)),
        ],
        out_specs=pl.BlockSpec((_BN, _H), lambda i: (i, 0)),
        out_shape=jax.ShapeDtypeStruct((_N, _H), jnp.float32),
        compiler_params=pltpu.CompilerParams(
            dimension_semantics=("arbitrary",)),
    )(accx, pa, pa, pb, pb, W1, W2, b2r)


def kernel(x, edge_index, edge_attr, u, batch, W1, b1, W2, b2):
    col2d = edge_index[1].astype(jnp.int32).reshape(_NBLK, _EB)
    zeros = jnp.zeros((_N, _H), jnp.float32)
    eaT = edge_attr.T
    ea1 = _tc_xpose(eaT, _XG1, 0)
    pa = _sc_scatter_add(_NCH1)(col2d[:_NCH1 * _CB],
                                ea1.reshape(_E1, _H), zeros)
    ea2 = _tc_xpose(eaT, _XG2, _XG1)
    pb = _sc_scatter_add(_NCH2)(col2d[_NCH1 * _CB:],
                                ea2.reshape(_E2, _H), zeros)
    batch3 = batch.astype(jnp.int32).reshape(_NB, 1, _BN)
    accx = _tc_mlp_x(x, batch3, u, W1, b1.reshape(1, _H))
    return _tc_mlp_fin(accx, pa, pb, W1, W2, b2.reshape(1, _H))
